# baseline (speedup 1.0000x reference)
import functools

import jax
import jax.numpy as jnp
from jax import lax
from jax.experimental import pallas as pl
from jax.experimental.pallas import tpu as pltpu
from jax.experimental.pallas import tpu_sc as plsc

_BLK = 128


def _rollm(x, d, axis):
    if axis == 1:
        return jnp.concatenate([x[:, d:], x[:, :d]], axis=1)
    return jnp.concatenate([x[d:, :], x[:d, :]], axis=0)


def _rollp(x, d, axis):
    n = x.shape[axis]
    if axis == 1:
        return jnp.concatenate([x[:, n - d:], x[:, :n - d]], axis=1)
    return jnp.concatenate([x[n - d:, :], x[:n - d, :]], axis=0)


def _bitonic_desc(key):
    r_dim, c_dim = key.shape
    n = r_dim * c_dim
    riota = lax.broadcasted_iota(jnp.int32, (r_dim, c_dim), 0)
    ciota = lax.broadcasted_iota(jnp.int32, (r_dim, c_dim), 1)
    pay = riota * c_dim + ciota
    k = 2
    while k <= n:
        d = k // 2
        while d >= 1:
            if d < c_dim:
                bd = (ciota & d) != 0
                pk = jnp.where(bd, _rollp(key, d, 1), _rollm(key, d, 1))
                pp = jnp.where(bd, _rollp(pay, d, 1), _rollm(pay, d, 1))
            else:
                dr = d // c_dim
                bd = (riota & dr) != 0
                pk = jnp.where(bd, _rollp(key, dr, 0), _rollm(key, dr, 0))
                pp = jnp.where(bd, _rollp(pay, dr, 0), _rollm(pay, dr, 0))
            if k < c_dim:
                bk = (ciota & k) != 0
            else:
                bk = (riota & (k // c_dim)) != 0
            take_min = bk ^ bd
            self_gt = (key > pk) | ((key == pk) & (pay < pp))
            use_partner = self_gt == take_min
            key = jnp.where(use_partner, pk, key)
            pay = jnp.where(use_partner, pp, pay)
            d //= 2
        k *= 2
    return key, pay


def _sort0_body(s_ref, h_ref, s0c_ref, spay_ref, hsc_ref):
    i = pl.program_id(0)

    @pl.when(i == 0)
    def _sort():
        _, spay = _bitonic_desc(lax.bitcast_convert_type(s_ref[...], jnp.int32))
        spay_ref[...] = spay

    @pl.when(i > 0)
    def _scale():
        hsc_ref[...] = h_ref[...] * s0c_ref[...]


def _sort0_scale(s, h):
    n, d = h.shape
    prev = lambda i: (jnp.maximum(i - 1, 0), 0)
    spay, hsc = pl.pallas_call(
        _sort0_body,
        grid=(n // _BLK + 1,),
        in_specs=[
            pl.BlockSpec((n // _BLK, _BLK), lambda i: (0, 0)),
            pl.BlockSpec((_BLK, d), prev),
            pl.BlockSpec((_BLK, 1), prev),
        ],
        out_specs=[
            pl.BlockSpec((n // _BLK, _BLK), lambda i: (0, 0)),
            pl.BlockSpec((_BLK, d), prev),
        ],
        out_shape=[
            jax.ShapeDtypeStruct((n // _BLK, _BLK), jnp.int32),
            jax.ShapeDtypeStruct((n, d), jnp.float32),
        ],
    )(s.reshape(n // _BLK, _BLK), h, s.reshape(n, 1))
    return spay.reshape(n), hsc


def _sc_gather(hsc, payload, k0):
    n, d = hsc.shape
    info = plsc.get_sparse_core_info()
    nc, ns, lanes = info.num_cores, info.num_subcores, info.num_lanes
    nw = nc * ns
    rpw = n // nw
    mesh = plsc.VectorSubcoreMesh(core_axis_name="c", subcore_axis_name="s")

    @functools.partial(
        pl.kernel,
        mesh=mesh,
        out_type=[
            jax.ShapeDtypeStruct((n, d), jnp.float32),
            jax.ShapeDtypeStruct((n,), jnp.int32),
        ],
        scratch_types=[
            pltpu.VMEM((rpw,), jnp.int32),
            pltpu.VMEM((rpw,), jnp.int32),
            pltpu.VMEM((rpw, d), jnp.float32),
            pltpu.SemaphoreType.DMA,
        ],
    )
    def gather_kernel(h_hbm, pay_hbm, flag_hbm, out_hbm, m0_hbm,
                      idx_v, m0_v, rows_v, sem):
        wid = lax.axis_index("s") * nc + lax.axis_index("c")
        base = wid * rpw
        pltpu.sync_copy(pay_hbm.at[pl.ds(base, rpw)], idx_v)
        pltpu.sync_copy(flag_hbm.at[pl.ds(base, rpw)], m0_v)
        pltpu.async_copy(h_hbm.at[idx_v], rows_v, sem).wait()
        pltpu.sync_copy(rows_v, out_hbm.at[pl.ds(base, rpw)])
        pltpu.async_copy(m0_v, m0_hbm.at[idx_v], sem).wait()

    flags = (jnp.arange(n, dtype=jnp.int32) < k0).astype(jnp.int32)
    return gather_kernel(hsc, payload, flags)


def _combine_body(k0, k1, s1_ref, hsc_ref, hp_ref, s1c_ref, m0c_ref,
                  out_ref, prm_ref):
    i = pl.program_id(0)

    @pl.when(i == 0)
    def _sort1():
        skey, spay = _bitonic_desc(lax.bitcast_convert_type(s1_ref[...], jnp.int32))
        r, c = (k1 - 1) // _BLK, (k1 - 1) % _BLK
        prm_ref[0] = skey[r, c]
        prm_ref[1] = spay[r, c]

    @pl.when(i > 0)
    def _combine():
        icol = (i - 1) * _BLK + lax.broadcasted_iota(jnp.int32, (_BLK, 1), 0)
        b1 = lax.bitcast_convert_type(s1c_ref[...], jnp.int32)
        m1 = (b1 > prm_ref[0]) | ((b1 == prm_ref[0]) & (icol <= prm_ref[1]))
        m0 = m0c_ref[...] != 0
        t0 = jnp.where(m0, hsc_ref[...], 0.0)
        t1 = jnp.where(m1, hp_ref[...] * s1c_ref[...], 0.0)
        out_ref[...] = t0 + t1


def _sort1_combine(s1p, hsc, hp0, m0, k0, k1):
    n, d = hsc.shape
    col = lambda x: x.reshape(n, 1)
    prev = lambda i: (jnp.maximum(i - 1, 0), 0)
    row_spec = pl.BlockSpec((_BLK, d), prev)
    col_spec = pl.BlockSpec((_BLK, 1), prev)
    return pl.pallas_call(
        functools.partial(_combine_body, k0, k1),
        grid=(n // _BLK + 1,),
        in_specs=[
            pl.BlockSpec((n // _BLK, _BLK), lambda i: (0, 0)),
            row_spec, row_spec, col_spec, col_spec,
        ],
        out_specs=pl.BlockSpec((_BLK, d), prev),
        out_shape=jax.ShapeDtypeStruct((n, d), jnp.float32),
        scratch_shapes=[pltpu.SMEM((2,), jnp.int32)],
    )(s1p.reshape(n // _BLK, _BLK), hsc, hp0, col(s1p), col(m0))


def kernel(g, h, proj_w0, proj_b0, proj_w1, proj_b1):
    n, d = h.shape
    k0 = max(2, int(0.8 * n))
    k1 = max(2, int(0.6 * k0))
    s0 = jax.nn.sigmoid(h @ proj_w0 + proj_b0[0])
    spay0, hsc = _sort0_scale(s0, h)
    hp0, m0 = _sc_gather(hsc, spay0, k0)
    s1 = jax.nn.sigmoid(hp0[:k0] @ proj_w1 + proj_b1[0])
    s1p = jnp.concatenate([s1, jnp.zeros((n - k0,), jnp.float32)])
    return _sort1_combine(s1p, hsc, hp0, m0, k0, k1)
